# baseline (speedup 1.0000x reference)
import functools

import jax
import jax.numpy as jnp
from jax import lax
from jax.experimental import pallas as pl
from jax.experimental.pallas import tpu as pltpu
from jax.experimental.pallas import tpu_sc as plsc

B = 3276800
V = 1000000
CHUNK = 6400
ROWS = V // 4
ROWS_T = 15600
ROWS_C = 400


def _rsqrt(x):
    i = lax.bitcast_convert_type(x, jnp.int32)
    i = jnp.int32(0x5F3759DF) - (i >> 1)
    y = lax.bitcast_convert_type(i, jnp.float32)
    for _ in range(3):
        y = y * (1.5 - 0.5 * x * y * y)
    return y


def _make_impl(b_per_w):
    n_chunks = b_per_w // CHUNK
    mesh = plsc.VectorSubcoreMesh(core_axis_name="c", subcore_axis_name="s")

    @functools.partial(
        pl.kernel,
        out_type=[jax.ShapeDtypeStruct((B,), jnp.float32),
                  jax.ShapeDtypeStruct((2, ROWS, 8), jnp.float32),
                  jax.ShapeDtypeStruct((2, ROWS, 8), jnp.float32)],
        mesh=mesh,
        compiler_params=pltpu.CompilerParams(
            needs_layout_passes=False, use_tc_tiling_on_sc=False),
        scratch_types=[
            pltpu.VMEM((CHUNK,), jnp.int32),
            pltpu.VMEM((CHUNK,), jnp.int32),
            pltpu.VMEM((CHUNK,), jnp.int32),
            pltpu.VMEM((CHUNK, 8), jnp.float32),
            pltpu.VMEM((CHUNK, 8), jnp.float32),
            pltpu.VMEM((CHUNK,), jnp.float32),
            pltpu.VMEM((64,), jnp.float32),
            pltpu.VMEM((16,), jnp.float32),
            pltpu.SemaphoreType.DMA,
        ],
    )
    def impl(legs_hbm, votes_hbm, ip_hbm, yes_hbm, no_hbm, w_hbm,
             out_hbm, ytbl, ntbl,
             votes_v, idx_v, legs_v, yrows_v, nrows_v, out_v, ip_v, w_v, sem):
        info = plsc.get_sparse_core_info()
        nc = info.num_cores
        scid = lax.axis_index("c")
        sid = lax.axis_index("s")
        wid = sid * nc + scid

        lanes = lax.iota(jnp.int32, 16)
        zeros = jnp.zeros((16,), jnp.int32)

        n_pass = (ROWS_T + jnp.where(sid == 15, ROWS - 16 * ROWS_T, 0)) // ROWS_C

        def pass_body(p, _):
            row0 = sid * ROWS_T + p * ROWS_C
            for src, dst in ((yes_hbm, ytbl), (no_hbm, ntbl)):
                pltpu.sync_copy(src.at[pl.ds(row0 * 8, ROWS_C * 8)],
                                out_v.at[pl.ds(0, ROWS_C * 8)])

                def body(i, _):
                    pos = lanes + i * 16
                    x = out_v[pl.ds(i * 16, 16)]
                    plsc.store_scatter(yrows_v, [pos >> 3, pos & 7], x)
                    return 0

                lax.fori_loop(0, ROWS_C // 2, body, 0, unroll=8)
                pltpu.sync_copy(yrows_v.at[pl.ds(0, ROWS_C), :],
                                dst.at[scid, pl.ds(row0, ROWS_C), :])
            return 0

        lax.fori_loop(0, n_pass, pass_body, 0)
        plsc.subcore_barrier()

        pltpu.sync_copy(w_hbm, w_v)
        pltpu.sync_copy(ip_hbm, ip_v)
        wv = w_v[...]
        w0 = wv.at[zeros].get(mode="promise_in_bounds")
        w1 = wv.at[zeros + 1].get(mode="promise_in_bounds")
        w20 = w0 * w0
        w21 = w1 * w1

        for j in range(2):
            rows = lanes + 16 * j
            i0 = rows * 2
            i1 = i0 + 1
            x0 = plsc.load_gather(ip_v, [i0])
            x1 = plsc.load_gather(ip_v, [i1])
            n2 = x0 * x0 + x1 * x1
            n = n2 * _rsqrt(n2)
            scale = jnp.where(n2 > 1.0, 1.0 / (n + 1e-7), 1.0)
            plsc.store_scatter(ip_v, [i0], x0 * scale)
            plsc.store_scatter(ip_v, [i1], x1 * scale)

        ysc = ytbl.at[scid]
        nsc = ntbl.at[scid]

        def chunk_body(g, _):
            base = wid * b_per_w + g * CHUNK
            pltpu.sync_copy(votes_hbm.at[pl.ds(base, CHUNK)], votes_v)

            def shift_body(i, _):
                k0 = i * 16
                idx_v[pl.ds(k0, 16)] = votes_v[pl.ds(k0, 16)] >> 2
                return 0

            lax.fori_loop(0, CHUNK // 16, shift_body, 0, unroll=4)
            cpy = pltpu.make_async_copy(ysc.at[idx_v], yrows_v, sem)
            cpn = pltpu.make_async_copy(nsc.at[idx_v], nrows_v, sem)
            cpy.start()
            cpn.start()
            pltpu.sync_copy(legs_hbm.at[pl.ds(base, CHUNK)], legs_v)
            cpy.wait()
            cpn.wait()

            def group_body(i, _):
                k0 = i * 16
                v = votes_v[pl.ds(k0, 16)]
                q2 = (v & 3) * 2
                lv = legs_v[pl.ds(k0, 16)]
                g0 = plsc.load_gather(ip_v, [lv * 2])
                g1 = plsc.load_gather(ip_v, [lv * 2 + 1])
                ridx = lanes + k0
                yp0 = plsc.load_gather(yrows_v, [ridx, q2])
                yp1 = plsc.load_gather(yrows_v, [ridx, q2 + 1])
                np0 = plsc.load_gather(nrows_v, [ridx, q2])
                np1 = plsc.load_gather(nrows_v, [ridx, q2 + 1])
                dy0 = g0 - yp0
                dy1 = g1 - yp1
                dn0 = g0 - np0
                dn1 = g1 - np1
                dy = w20 * (dy0 * dy0) + w21 * (dy1 * dy1)
                dn = w20 * (dn0 * dn0) + w21 * (dn1 * dn1)
                out_v[pl.ds(k0, 16)] = jnp.exp(-0.5 * dy) - jnp.exp(-0.5 * dn)
                return 0

            lax.fori_loop(0, CHUNK // 16, group_body, 0, unroll=4)
            pltpu.sync_copy(out_v, out_hbm.at[pl.ds(base, CHUNK)])
            return 0

        lax.fori_loop(0, n_chunks, chunk_body, 0)

    return impl


def kernel(legs, votes, ideal_points, yes_points, no_points, w):
    yes_flat = yes_points.reshape(-1)
    no_flat = no_points.reshape(-1)
    wpad = jnp.pad(w.astype(jnp.float32), (0, 14))
    ip_flat = jnp.reshape(ideal_points.astype(jnp.float32), (-1,))
    info = plsc.get_sparse_core_info()
    nw = info.num_cores * info.num_subcores
    impl = _make_impl(B // nw)
    out, _, _ = impl(legs.astype(jnp.int32), votes.astype(jnp.int32),
                     ip_flat, yes_flat, no_flat, wpad)
    return out


# speedup vs baseline: 4.4162x; 4.4162x over previous
import functools

import jax
import jax.numpy as jnp
from jax import lax
from jax.experimental import pallas as pl
from jax.experimental.pallas import tpu as pltpu
from jax.experimental.pallas import tpu_sc as plsc

B = 3276800
V = 1000000
CHUNK = 6400
ROWS = V // 4
GBLK = 999936 // 128
BLK_W = 488
PASS = 16


def _rsqrt(x):
    i = lax.bitcast_convert_type(x, jnp.int32)
    i = jnp.int32(0x5F3759DF) - (i >> 1)
    y = lax.bitcast_convert_type(i, jnp.float32)
    for _ in range(3):
        y = y * (1.5 - 0.5 * x * y * y)
    return y


def _make_impl(b_per_w):
    n_chunks = b_per_w // CHUNK
    mesh = plsc.VectorSubcoreMesh(core_axis_name="c", subcore_axis_name="s")

    @functools.partial(
        pl.kernel,
        out_type=[jax.ShapeDtypeStruct((B,), jnp.float32),
                  jax.ShapeDtypeStruct((2, ROWS, 8), jnp.float32),
                  jax.ShapeDtypeStruct((2, ROWS, 8), jnp.float32)],
        mesh=mesh,
        compiler_params=pltpu.CompilerParams(
            needs_layout_passes=False, use_tc_tiling_on_sc=False),
        scratch_types=[
            pltpu.VMEM((CHUNK,), jnp.int32),
            pltpu.VMEM((CHUNK,), jnp.int32),
            pltpu.VMEM((CHUNK,), jnp.int32),
            pltpu.VMEM((CHUNK, 8), jnp.float32),
            pltpu.VMEM((CHUNK, 8), jnp.float32),
            pltpu.VMEM((CHUNK,), jnp.float32),
            pltpu.VMEM((64,), jnp.float32),
            pltpu.VMEM((16,), jnp.float32),
            pltpu.SemaphoreType.DMA,
        ],
    )
    def impl(legs_hbm, votes_hbm, ip_hbm, ymain_hbm, ytail_hbm, nmain_hbm,
             ntail_hbm, w_hbm, out_hbm, ytbl, ntbl,
             votes_v, idx_v, legs_v, yrows_v, nrows_v, out_v, ip_v, w_v, sem):
        info = plsc.get_sparse_core_info()
        nc = info.num_cores
        scid = lax.axis_index("c")
        sid = lax.axis_index("s")
        wid = sid * nc + scid

        lanes = lax.iota(jnp.int32, 16)
        zeros = jnp.zeros((16,), jnp.int32)

        def do_pass(src_ref, dst_tbl, b, p_blocks):
            elems = 256 * p_blocks
            pltpu.sync_copy(src_ref.at[pl.ds(b * 256, elems)],
                            out_v.at[pl.ds(0, elems)])

            def body(i, _):
                s = lanes + i * 16
                blk = s >> 8
                q = s & 255
                c = q >> 7
                k = q & 127
                row = blk * 32 + (k >> 2)
                col = 2 * (k & 3) + c
                plsc.store_scatter(yrows_v, [row, col],
                                   out_v[pl.ds(i * 16, 16)])
                return 0

            lax.fori_loop(0, elems // 16, body, 0, unroll=8)
            pltpu.sync_copy(yrows_v.at[pl.ds(0, 32 * p_blocks), :],
                            dst_tbl.at[scid, pl.ds(b * 32, 32 * p_blocks), :])

        b0 = sid * BLK_W + jnp.minimum(sid, 4)
        for src_ref, dst_tbl in ((ymain_hbm, ytbl), (nmain_hbm, ntbl)):
            def pass_body(p, _):
                do_pass(src_ref, dst_tbl, b0 + p * PASS, PASS)
                return 0

            lax.fori_loop(0, BLK_W // PASS, pass_body, 0)
            do_pass(src_ref, dst_tbl, b0 + (BLK_W // PASS) * PASS,
                    BLK_W - (BLK_W // PASS) * PASS)

            @pl.when(sid < 4)
            def _():
                do_pass(src_ref, dst_tbl, b0 + BLK_W, 1)

        @pl.when(sid == 15)
        def _():
            for src_ref, dst_tbl in ((ytail_hbm, ytbl), (ntail_hbm, ntbl)):
                pltpu.sync_copy(src_ref, out_v.at[pl.ds(0, 128)])
                for i in range(8):
                    pos = lanes + i * 16
                    plsc.store_scatter(yrows_v, [pos >> 3, pos & 7],
                                       out_v[pl.ds(i * 16, 16)])
                pltpu.sync_copy(yrows_v.at[pl.ds(0, 16), :],
                                dst_tbl.at[scid, pl.ds(GBLK * 32, 16), :])

        plsc.subcore_barrier()

        pltpu.sync_copy(w_hbm, w_v)
        pltpu.sync_copy(ip_hbm, ip_v)
        wv = w_v[...]
        w0 = wv.at[zeros].get(mode="promise_in_bounds")
        w1 = wv.at[zeros + 1].get(mode="promise_in_bounds")
        w20 = w0 * w0
        w21 = w1 * w1

        for j in range(2):
            rows = lanes + 16 * j
            i0 = rows * 2
            i1 = i0 + 1
            x0 = plsc.load_gather(ip_v, [i0])
            x1 = plsc.load_gather(ip_v, [i1])
            n2 = x0 * x0 + x1 * x1
            n = n2 * _rsqrt(n2)
            scale = jnp.where(n2 > 1.0, 1.0 / (n + 1e-7), 1.0)
            plsc.store_scatter(ip_v, [i0], x0 * scale)
            plsc.store_scatter(ip_v, [i1], x1 * scale)

        ysc = ytbl.at[scid]
        nsc = ntbl.at[scid]

        def chunk_body(g, _):
            base = wid * b_per_w + g * CHUNK
            pltpu.sync_copy(votes_hbm.at[pl.ds(base, CHUNK)], votes_v)

            def shift_body(i, _):
                k0 = i * 16
                idx_v[pl.ds(k0, 16)] = votes_v[pl.ds(k0, 16)] >> 2
                return 0

            lax.fori_loop(0, CHUNK // 16, shift_body, 0, unroll=4)
            cpy = pltpu.make_async_copy(ysc.at[idx_v], yrows_v, sem)
            cpn = pltpu.make_async_copy(nsc.at[idx_v], nrows_v, sem)
            cpy.start()
            cpn.start()
            pltpu.sync_copy(legs_hbm.at[pl.ds(base, CHUNK)], legs_v)
            cpy.wait()
            cpn.wait()

            def group_body(i, _):
                k0 = i * 16
                v = votes_v[pl.ds(k0, 16)]
                q2 = (v & 3) * 2
                lv = legs_v[pl.ds(k0, 16)]
                g0 = plsc.load_gather(ip_v, [lv * 2])
                g1 = plsc.load_gather(ip_v, [lv * 2 + 1])
                ridx = lanes + k0
                yp0 = plsc.load_gather(yrows_v, [ridx, q2])
                yp1 = plsc.load_gather(yrows_v, [ridx, q2 + 1])
                np0 = plsc.load_gather(nrows_v, [ridx, q2])
                np1 = plsc.load_gather(nrows_v, [ridx, q2 + 1])
                dy0 = g0 - yp0
                dy1 = g1 - yp1
                dn0 = g0 - np0
                dn1 = g1 - np1
                dy = w20 * (dy0 * dy0) + w21 * (dy1 * dy1)
                dn = w20 * (dn0 * dn0) + w21 * (dn1 * dn1)
                out_v[pl.ds(k0, 16)] = jnp.exp(-0.5 * dy) - jnp.exp(-0.5 * dn)
                return 0

            lax.fori_loop(0, CHUNK // 16, group_body, 0, unroll=4)
            pltpu.sync_copy(out_v, out_hbm.at[pl.ds(base, CHUNK)])
            return 0

        lax.fori_loop(0, n_chunks, chunk_body, 0)

    return impl


def kernel(legs, votes, ideal_points, yes_points, no_points, w):
    M = GBLK * 128
    ymain = yes_points[:M].reshape(GBLK, 128, 2).transpose(0, 2, 1).reshape(-1)
    nmain = no_points[:M].reshape(GBLK, 128, 2).transpose(0, 2, 1).reshape(-1)
    ytail = yes_points[M:].reshape(-1)
    ntail = no_points[M:].reshape(-1)
    wpad = jnp.pad(w.astype(jnp.float32), (0, 14))
    ip_flat = jnp.reshape(ideal_points.astype(jnp.float32), (-1,))
    info = plsc.get_sparse_core_info()
    nw = info.num_cores * info.num_subcores
    impl = _make_impl(B // nw)
    out, _, _ = impl(legs.astype(jnp.int32), votes.astype(jnp.int32),
                     ip_flat, ymain, ytail, nmain, ntail, wpad)
    return out


# speedup vs baseline: 5.7692x; 1.3064x over previous
import functools

import jax
import jax.numpy as jnp
from jax import lax
from jax.experimental import pallas as pl
from jax.experimental.pallas import tpu as pltpu
from jax.experimental.pallas import tpu_sc as plsc

B = 3276800
V = 1000000
CHUNK = 3200
ROWS = V // 4
GBLK = 999936 // 128
BLK_W = 488
PASS = 8


def _rsqrt(x):
    i = lax.bitcast_convert_type(x, jnp.int32)
    i = jnp.int32(0x5F3759DF) - (i >> 1)
    y = lax.bitcast_convert_type(i, jnp.float32)
    for _ in range(3):
        y = y * (1.5 - 0.5 * x * y * y)
    return y


def _make_impl(b_per_w):
    n_chunks = b_per_w // CHUNK
    assert n_chunks % 2 == 0
    mesh = plsc.VectorSubcoreMesh(core_axis_name="c", subcore_axis_name="s")

    @functools.partial(
        pl.kernel,
        out_type=[jax.ShapeDtypeStruct((B,), jnp.float32),
                  jax.ShapeDtypeStruct((2, ROWS, 8), jnp.float32),
                  jax.ShapeDtypeStruct((2, ROWS, 8), jnp.float32)],
        mesh=mesh,
        compiler_params=pltpu.CompilerParams(
            needs_layout_passes=False, use_tc_tiling_on_sc=False),
        scratch_types=[
            pltpu.VMEM((CHUNK,), jnp.int32),
            pltpu.VMEM((CHUNK,), jnp.int32),
            pltpu.VMEM((CHUNK,), jnp.int32),
            pltpu.VMEM((CHUNK,), jnp.int32),
            pltpu.VMEM((CHUNK,), jnp.int32),
            pltpu.VMEM((CHUNK,), jnp.int32),
            pltpu.VMEM((CHUNK, 8), jnp.float32),
            pltpu.VMEM((CHUNK, 8), jnp.float32),
            pltpu.VMEM((CHUNK, 8), jnp.float32),
            pltpu.VMEM((CHUNK, 8), jnp.float32),
            pltpu.VMEM((CHUNK,), jnp.float32),
            pltpu.VMEM((CHUNK,), jnp.float32),
            pltpu.VMEM((64,), jnp.float32),
            pltpu.VMEM((16,), jnp.float32),
            pltpu.SemaphoreType.DMA,
            pltpu.SemaphoreType.DMA,
        ],
    )
    def impl(legs_hbm, votes_hbm, ip_hbm, ymain_hbm, ytail_hbm, nmain_hbm,
             ntail_hbm, w_hbm, out_hbm, ytbl, ntbl,
             votes_v0, votes_v1, idx_v0, idx_v1, legs_v0, legs_v1,
             yrows_v0, yrows_v1, nrows_v0, nrows_v1, out_v0, out_v1,
             ip_v, w_v, sem0, sem1):
        info = plsc.get_sparse_core_info()
        nc = info.num_cores
        scid = lax.axis_index("c")
        sid = lax.axis_index("s")
        wid = sid * nc + scid

        votes_b = (votes_v0, votes_v1)
        idx_b = (idx_v0, idx_v1)
        legs_b = (legs_v0, legs_v1)
        yrows_b = (yrows_v0, yrows_v1)
        nrows_b = (nrows_v0, nrows_v1)
        out_b = (out_v0, out_v1)
        sem_b = (sem0, sem1)

        lanes = lax.iota(jnp.int32, 16)
        zeros = jnp.zeros((16,), jnp.int32)

        def do_pass(src_ref, dst_tbl, b, p_blocks):
            elems = 256 * p_blocks
            pltpu.sync_copy(src_ref.at[pl.ds(b * 256, elems)],
                            out_v0.at[pl.ds(0, elems)])

            def body(i, _):
                s = lanes + i * 16
                blk = s >> 8
                q = s & 255
                c = q >> 7
                k = q & 127
                row = blk * 32 + (k >> 2)
                col = 2 * (k & 3) + c
                plsc.store_scatter(yrows_v0, [row, col],
                                   out_v0[pl.ds(i * 16, 16)])
                return 0

            lax.fori_loop(0, elems // 16, body, 0, unroll=8)
            pltpu.sync_copy(yrows_v0.at[pl.ds(0, 32 * p_blocks), :],
                            dst_tbl.at[scid, pl.ds(b * 32, 32 * p_blocks), :])

        b0 = sid * BLK_W + jnp.minimum(sid, 4)
        for src_ref, dst_tbl in ((ymain_hbm, ytbl), (nmain_hbm, ntbl)):
            def pass_body(p, _):
                do_pass(src_ref, dst_tbl, b0 + p * PASS, PASS)
                return 0

            lax.fori_loop(0, BLK_W // PASS, pass_body, 0)

            @pl.when(sid < 4)
            def _():
                do_pass(src_ref, dst_tbl, b0 + BLK_W, 1)

        @pl.when(sid == 15)
        def _():
            for src_ref, dst_tbl in ((ytail_hbm, ytbl), (ntail_hbm, ntbl)):
                pltpu.sync_copy(src_ref, out_v0.at[pl.ds(0, 128)])
                for i in range(8):
                    pos = lanes + i * 16
                    plsc.store_scatter(yrows_v0, [pos >> 3, pos & 7],
                                       out_v0[pl.ds(i * 16, 16)])
                pltpu.sync_copy(yrows_v0.at[pl.ds(0, 16), :],
                                dst_tbl.at[scid, pl.ds(GBLK * 32, 16), :])

        plsc.subcore_barrier()

        pltpu.sync_copy(w_hbm, w_v)
        pltpu.sync_copy(ip_hbm, ip_v)
        wv = w_v[...]
        w0 = wv.at[zeros].get(mode="promise_in_bounds")
        w1 = wv.at[zeros + 1].get(mode="promise_in_bounds")
        w20 = w0 * w0
        w21 = w1 * w1

        for j in range(2):
            rows = lanes + 16 * j
            i0 = rows * 2
            i1 = i0 + 1
            x0 = plsc.load_gather(ip_v, [i0])
            x1 = plsc.load_gather(ip_v, [i1])
            n2 = x0 * x0 + x1 * x1
            n = n2 * _rsqrt(n2)
            scale = jnp.where(n2 > 1.0, 1.0 / (n + 1e-7), 1.0)
            plsc.store_scatter(ip_v, [i0], x0 * scale)
            plsc.store_scatter(ip_v, [i1], x1 * scale)

        ysc = ytbl.at[scid]
        nsc = ntbl.at[scid]
        wbase = wid * b_per_w

        def fetch(g, b):
            base = wbase + g * CHUNK
            pltpu.sync_copy(votes_hbm.at[pl.ds(base, CHUNK)], votes_b[b])

            def shift_body(i, _):
                k0 = i * 16
                idx_b[b][pl.ds(k0, 16)] = votes_b[b][pl.ds(k0, 16)] >> 2
                return 0

            lax.fori_loop(0, CHUNK // 16, shift_body, 0, unroll=4)
            pltpu.make_async_copy(ysc.at[idx_b[b]], yrows_b[b], sem_b[b]).start()
            pltpu.make_async_copy(nsc.at[idx_b[b]], nrows_b[b], sem_b[b]).start()
            pltpu.sync_copy(legs_hbm.at[pl.ds(base, CHUNK)], legs_b[b])

        def compute(g, b):
            pltpu.make_async_copy(ysc.at[idx_b[b]], yrows_b[b], sem_b[b]).wait()
            pltpu.make_async_copy(nsc.at[idx_b[b]], nrows_b[b], sem_b[b]).wait()

            def group_body(i, _):
                k0 = i * 16
                v = votes_b[b][pl.ds(k0, 16)]
                q2 = (v & 3) * 2
                lv = legs_b[b][pl.ds(k0, 16)]
                g0 = plsc.load_gather(ip_v, [lv * 2])
                g1 = plsc.load_gather(ip_v, [lv * 2 + 1])
                ridx = lanes + k0
                yp0 = plsc.load_gather(yrows_b[b], [ridx, q2])
                yp1 = plsc.load_gather(yrows_b[b], [ridx, q2 + 1])
                np0 = plsc.load_gather(nrows_b[b], [ridx, q2])
                np1 = plsc.load_gather(nrows_b[b], [ridx, q2 + 1])
                dy0 = g0 - yp0
                dy1 = g1 - yp1
                dn0 = g0 - np0
                dn1 = g1 - np1
                dy = w20 * (dy0 * dy0) + w21 * (dy1 * dy1)
                dn = w20 * (dn0 * dn0) + w21 * (dn1 * dn1)
                out_b[b][pl.ds(k0, 16)] = (jnp.exp(-0.5 * dy)
                                           - jnp.exp(-0.5 * dn))
                return 0

            lax.fori_loop(0, CHUNK // 16, group_body, 0, unroll=4)
            pltpu.sync_copy(out_b[b], out_hbm.at[pl.ds(wbase + g * CHUNK,
                                                       CHUNK)])

        fetch(0, 0)

        def loop_body(gg, _):
            for b in range(2):
                g = gg * 2 + b

                @pl.when(g + 1 < n_chunks)
                def _():
                    fetch(g + 1, 1 - b)

                compute(g, b)
            return 0

        lax.fori_loop(0, n_chunks // 2, loop_body, 0)

    return impl


def kernel(legs, votes, ideal_points, yes_points, no_points, w):
    M = GBLK * 128
    ymain = yes_points[:M].reshape(GBLK, 128, 2).transpose(0, 2, 1).reshape(-1)
    nmain = no_points[:M].reshape(GBLK, 128, 2).transpose(0, 2, 1).reshape(-1)
    ytail = yes_points[M:].reshape(-1)
    ntail = no_points[M:].reshape(-1)
    wpad = jnp.pad(w.astype(jnp.float32), (0, 14))
    ip_flat = jnp.reshape(ideal_points.astype(jnp.float32), (-1,))
    info = plsc.get_sparse_core_info()
    nw = info.num_cores * info.num_subcores
    impl = _make_impl(B // nw)
    out, _, _ = impl(legs.astype(jnp.int32), votes.astype(jnp.int32),
                     ip_flat, ymain, ytail, nmain, ntail, wpad)
    return out


# speedup vs baseline: 6.1599x; 1.0677x over previous
import functools

import jax
import jax.numpy as jnp
from jax import lax
from jax.experimental import pallas as pl
from jax.experimental.pallas import tpu as pltpu
from jax.experimental.pallas import tpu_sc as plsc

B = 3276800
V = 1000000
CHUNK = 5120
ROWS = V // 2
GBLK = 999936 // 128
BLK_W = 488
PASS = 8


def _rsqrt(x):
    i = lax.bitcast_convert_type(x, jnp.int32)
    i = jnp.int32(0x5F3759DF) - (i >> 1)
    y = lax.bitcast_convert_type(i, jnp.float32)
    for _ in range(3):
        y = y * (1.5 - 0.5 * x * y * y)
    return y


def _make_impl(b_per_w):
    n_chunks = b_per_w // CHUNK
    assert n_chunks % 2 == 0
    mesh = plsc.VectorSubcoreMesh(core_axis_name="c", subcore_axis_name="s")

    @functools.partial(
        pl.kernel,
        out_type=[jax.ShapeDtypeStruct((B,), jnp.float32),
                  jax.ShapeDtypeStruct((2, ROWS, 8), jnp.float32)],
        mesh=mesh,
        compiler_params=pltpu.CompilerParams(
            needs_layout_passes=False, use_tc_tiling_on_sc=False),
        scratch_types=[
            pltpu.VMEM((CHUNK,), jnp.int32),
            pltpu.VMEM((CHUNK,), jnp.int32),
            pltpu.VMEM((CHUNK,), jnp.int32),
            pltpu.VMEM((CHUNK,), jnp.int32),
            pltpu.VMEM((CHUNK,), jnp.int32),
            pltpu.VMEM((CHUNK,), jnp.int32),
            pltpu.VMEM((CHUNK, 8), jnp.float32),
            pltpu.VMEM((CHUNK, 8), jnp.float32),
            pltpu.VMEM((CHUNK,), jnp.float32),
            pltpu.VMEM((CHUNK,), jnp.float32),
            pltpu.VMEM((64,), jnp.float32),
            pltpu.VMEM((16,), jnp.float32),
            pltpu.SemaphoreType.DMA,
            pltpu.SemaphoreType.DMA,
        ],
    )
    def impl(legs_hbm, votes_hbm, ip_hbm, ymain_hbm, ytail_hbm, nmain_hbm,
             ntail_hbm, w_hbm, out_hbm, stbl,
             votes_v0, votes_v1, idx_v0, idx_v1, legs_v0, legs_v1,
             rows_v0, rows_v1, out_v0, out_v1, ip_v, w_v, sem0, sem1):
        info = plsc.get_sparse_core_info()
        nc = info.num_cores
        scid = lax.axis_index("c")
        sid = lax.axis_index("s")
        wid = sid * nc + scid

        votes_b = (votes_v0, votes_v1)
        idx_b = (idx_v0, idx_v1)
        legs_b = (legs_v0, legs_v1)
        rows_b = (rows_v0, rows_v1)
        out_b = (out_v0, out_v1)
        sem_b = (sem0, sem1)

        lanes = lax.iota(jnp.int32, 16)
        zeros = jnp.zeros((16,), jnp.int32)

        def scatter_block(src_v, elems, coff):
            def body(i, _):
                s = lanes + i * 16
                blk = s >> 8
                q = s & 255
                c = q >> 7
                k = q & 127
                row = blk * 64 + (k >> 1)
                col = 2 * (k & 1) + c + coff
                plsc.store_scatter(rows_v0, [row, col],
                                   src_v[pl.ds(i * 16, 16)])
                return 0

            lax.fori_loop(0, elems // 16, body, 0, unroll=8)

        def do_pass(b, p_blocks):
            elems = 256 * p_blocks
            for src_ref, coff in ((ymain_hbm, 0), (nmain_hbm, 4)):
                pltpu.sync_copy(src_ref.at[pl.ds(b * 256, elems)],
                                out_v0.at[pl.ds(0, elems)])
                scatter_block(out_v0, elems, coff)
            pltpu.sync_copy(rows_v0.at[pl.ds(0, 64 * p_blocks), :],
                            stbl.at[scid, pl.ds(b * 64, 64 * p_blocks), :])

        b0 = sid * BLK_W + jnp.minimum(sid, 4)

        def pass_body(p, _):
            do_pass(b0 + p * PASS, PASS)
            return 0

        lax.fori_loop(0, BLK_W // PASS, pass_body, 0)

        @pl.when(sid < 4)
        def _():
            do_pass(b0 + BLK_W, 1)

        @pl.when(sid == 15)
        def _():
            for src_ref, coff in ((ytail_hbm, 0), (ntail_hbm, 4)):
                pltpu.sync_copy(src_ref, out_v0.at[pl.ds(0, 128)])
                for i in range(8):
                    pos = lanes + i * 16
                    row = pos >> 2
                    col = (pos >> 1) % 2 * 2 + (pos & 1) + coff
                    plsc.store_scatter(rows_v0, [row, col],
                                       out_v0[pl.ds(i * 16, 16)])
            pltpu.sync_copy(rows_v0.at[pl.ds(0, 32), :],
                            stbl.at[scid, pl.ds(GBLK * 64, 32), :])

        plsc.subcore_barrier()

        pltpu.sync_copy(w_hbm, w_v)
        pltpu.sync_copy(ip_hbm, ip_v)
        wv = w_v[...]
        w0 = wv.at[zeros].get(mode="promise_in_bounds")
        w1 = wv.at[zeros + 1].get(mode="promise_in_bounds")
        w20 = w0 * w0
        w21 = w1 * w1

        for j in range(2):
            rows = lanes + 16 * j
            i0 = rows * 2
            i1 = i0 + 1
            x0 = plsc.load_gather(ip_v, [i0])
            x1 = plsc.load_gather(ip_v, [i1])
            n2 = x0 * x0 + x1 * x1
            n = n2 * _rsqrt(n2)
            scale = jnp.where(n2 > 1.0, 1.0 / (n + 1e-7), 1.0)
            plsc.store_scatter(ip_v, [i0], x0 * scale)
            plsc.store_scatter(ip_v, [i1], x1 * scale)

        ssc = stbl.at[scid]
        wbase = wid * b_per_w

        def fetch(g, b):
            base = wbase + g * CHUNK
            pltpu.sync_copy(votes_hbm.at[pl.ds(base, CHUNK)], votes_b[b])

            def shift_body(i, _):
                k0 = i * 16
                idx_b[b][pl.ds(k0, 16)] = votes_b[b][pl.ds(k0, 16)] >> 1
                return 0

            lax.fori_loop(0, CHUNK // 16, shift_body, 0, unroll=4)
            pltpu.make_async_copy(ssc.at[idx_b[b]], rows_b[b], sem_b[b]).start()
            pltpu.sync_copy(legs_hbm.at[pl.ds(base, CHUNK)], legs_b[b])

        def compute(g, b):
            pltpu.make_async_copy(ssc.at[idx_b[b]], rows_b[b], sem_b[b]).wait()

            def group_body(i, _):
                k0 = i * 16
                v = votes_b[b][pl.ds(k0, 16)]
                p2 = (v & 1) * 2
                lv = legs_b[b][pl.ds(k0, 16)]
                g0 = plsc.load_gather(ip_v, [lv * 2])
                g1 = plsc.load_gather(ip_v, [lv * 2 + 1])
                ridx = lanes + k0
                yp0 = plsc.load_gather(rows_b[b], [ridx, p2])
                yp1 = plsc.load_gather(rows_b[b], [ridx, p2 + 1])
                np0 = plsc.load_gather(rows_b[b], [ridx, p2 + 4])
                np1 = plsc.load_gather(rows_b[b], [ridx, p2 + 5])
                dy0 = g0 - yp0
                dy1 = g1 - yp1
                dn0 = g0 - np0
                dn1 = g1 - np1
                dy = w20 * (dy0 * dy0) + w21 * (dy1 * dy1)
                dn = w20 * (dn0 * dn0) + w21 * (dn1 * dn1)
                out_b[b][pl.ds(k0, 16)] = (jnp.exp(-0.5 * dy)
                                           - jnp.exp(-0.5 * dn))
                return 0

            lax.fori_loop(0, CHUNK // 16, group_body, 0, unroll=8)
            pltpu.sync_copy(out_b[b], out_hbm.at[pl.ds(wbase + g * CHUNK,
                                                       CHUNK)])

        fetch(0, 0)

        def loop_body(gg, _):
            for b in range(2):
                g = gg * 2 + b

                @pl.when(g + 1 < n_chunks)
                def _():
                    fetch(g + 1, 1 - b)

                compute(g, b)
            return 0

        lax.fori_loop(0, n_chunks // 2, loop_body, 0)

    return impl


def kernel(legs, votes, ideal_points, yes_points, no_points, w):
    M = GBLK * 128
    ymain = yes_points[:M].reshape(GBLK, 128, 2).transpose(0, 2, 1).reshape(-1)
    nmain = no_points[:M].reshape(GBLK, 128, 2).transpose(0, 2, 1).reshape(-1)
    ytail = yes_points[M:].reshape(-1)
    ntail = no_points[M:].reshape(-1)
    wpad = jnp.pad(w.astype(jnp.float32), (0, 14))
    ip_flat = jnp.reshape(ideal_points.astype(jnp.float32), (-1,))
    info = plsc.get_sparse_core_info()
    nw = info.num_cores * info.num_subcores
    impl = _make_impl(B // nw)
    out, _ = impl(legs.astype(jnp.int32), votes.astype(jnp.int32),
                  ip_flat, ymain, ytail, nmain, ntail, wpad)
    return out


# speedup vs baseline: 8.8634x; 1.4389x over previous
import functools

import jax
import jax.numpy as jnp
from jax import lax
from jax.experimental import pallas as pl
from jax.experimental.pallas import tpu as pltpu
from jax.experimental.pallas import tpu_sc as plsc

B = 3276800
V = 1000000
CHUNK = 5120
ROWS = V // 2
GBLK = 999936 // 128
BLK_W = 488
PASS = 8


def _rsqrt(x):
    i = lax.bitcast_convert_type(x, jnp.int32)
    i = jnp.int32(0x5F3759DF) - (i >> 1)
    y = lax.bitcast_convert_type(i, jnp.float32)
    for _ in range(3):
        y = y * (1.5 - 0.5 * x * y * y)
    return y


def _make_impl(b_per_w):
    n_chunks = b_per_w // CHUNK
    assert n_chunks % 2 == 0
    mesh = plsc.VectorSubcoreMesh(core_axis_name="c", subcore_axis_name="s")

    @functools.partial(
        pl.kernel,
        out_type=[jax.ShapeDtypeStruct((B,), jnp.float32),
                  jax.ShapeDtypeStruct((2, ROWS, 8), jnp.float32)],
        mesh=mesh,
        compiler_params=pltpu.CompilerParams(
            needs_layout_passes=False, use_tc_tiling_on_sc=False),
        scratch_types=[
            pltpu.VMEM((CHUNK,), jnp.int32),
            pltpu.VMEM((CHUNK,), jnp.int32),
            pltpu.VMEM((CHUNK,), jnp.int32),
            pltpu.VMEM((CHUNK,), jnp.int32),
            pltpu.VMEM((CHUNK,), jnp.int32),
            pltpu.VMEM((CHUNK,), jnp.int32),
            pltpu.VMEM((CHUNK, 8), jnp.float32),
            pltpu.VMEM((CHUNK, 8), jnp.float32),
            pltpu.VMEM((CHUNK,), jnp.float32),
            pltpu.VMEM((CHUNK,), jnp.float32),
            pltpu.VMEM((64,), jnp.float32),
            pltpu.VMEM((16,), jnp.float32),
            pltpu.SemaphoreType.DMA,
            pltpu.SemaphoreType.DMA,
        ],
    )
    def impl(legs_hbm, votes_hbm, ip_hbm, ymain_hbm, ytail_hbm, nmain_hbm,
             ntail_hbm, w_hbm, out_hbm, stbl,
             votes_v0, votes_v1, idx_v0, idx_v1, legs_v0, legs_v1,
             rows_v0, rows_v1, out_v0, out_v1, ip_v, w_v, sem0, sem1):
        info = plsc.get_sparse_core_info()
        nc = info.num_cores
        scid = lax.axis_index("c")
        sid = lax.axis_index("s")
        wid = sid * nc + scid

        votes_b = (votes_v0, votes_v1)
        idx_b = (idx_v0, idx_v1)
        legs_b = (legs_v0, legs_v1)
        rows_b = (rows_v0, rows_v1)
        out_b = (out_v0, out_v1)
        sem_b = (sem0, sem1)

        lanes = lax.iota(jnp.int32, 16)
        zeros = jnp.zeros((16,), jnp.int32)

        def scatter_block(src_v, elems, coff):
            @plsc.parallel_loop(0, elems // 16, unroll=8)
            def body(i):
                s = lanes + i * 16
                blk = s >> 8
                q = s & 255
                c = q >> 7
                k = q & 127
                row = blk * 64 + (k >> 1)
                col = 2 * (k & 1) + c + coff
                plsc.store_scatter(rows_v0, [row, col],
                                   src_v[pl.ds(i * 16, 16)])

        def do_pass(b, p_blocks):
            elems = 256 * p_blocks
            for src_ref, coff in ((ymain_hbm, 0), (nmain_hbm, 4)):
                pltpu.sync_copy(src_ref.at[pl.ds(b * 256, elems)],
                                out_v0.at[pl.ds(0, elems)])
                scatter_block(out_v0, elems, coff)
            pltpu.sync_copy(rows_v0.at[pl.ds(0, 64 * p_blocks), :],
                            stbl.at[scid, pl.ds(b * 64, 64 * p_blocks), :])

        b0 = sid * BLK_W + jnp.minimum(sid, 4)

        def pass_body(p, _):
            do_pass(b0 + p * PASS, PASS)
            return 0

        lax.fori_loop(0, BLK_W // PASS, pass_body, 0)

        @pl.when(sid < 4)
        def _():
            do_pass(b0 + BLK_W, 1)

        @pl.when(sid == 15)
        def _():
            for src_ref, coff in ((ytail_hbm, 0), (ntail_hbm, 4)):
                pltpu.sync_copy(src_ref, out_v0.at[pl.ds(0, 128)])
                for i in range(8):
                    pos = lanes + i * 16
                    row = pos >> 2
                    col = (pos >> 1) % 2 * 2 + (pos & 1) + coff
                    plsc.store_scatter(rows_v0, [row, col],
                                       out_v0[pl.ds(i * 16, 16)])
            pltpu.sync_copy(rows_v0.at[pl.ds(0, 32), :],
                            stbl.at[scid, pl.ds(GBLK * 64, 32), :])

        plsc.subcore_barrier()

        pltpu.sync_copy(w_hbm, w_v)
        pltpu.sync_copy(ip_hbm, ip_v)
        wv = w_v[...]
        w0 = wv.at[zeros].get(mode="promise_in_bounds")
        w1 = wv.at[zeros + 1].get(mode="promise_in_bounds")
        w20 = w0 * w0
        w21 = w1 * w1

        for j in range(2):
            rows = lanes + 16 * j
            i0 = rows * 2
            i1 = i0 + 1
            x0 = plsc.load_gather(ip_v, [i0])
            x1 = plsc.load_gather(ip_v, [i1])
            n2 = x0 * x0 + x1 * x1
            n = n2 * _rsqrt(n2)
            scale = jnp.where(n2 > 1.0, 1.0 / (n + 1e-7), 1.0)
            plsc.store_scatter(ip_v, [i0], x0 * scale)
            plsc.store_scatter(ip_v, [i1], x1 * scale)

        ssc = stbl.at[scid]
        wbase = wid * b_per_w

        def fetch(g, b):
            base = wbase + g * CHUNK
            pltpu.sync_copy(votes_hbm.at[pl.ds(base, CHUNK)], votes_b[b])

            @plsc.parallel_loop(0, CHUNK // 16, unroll=8)
            def shift_body(i):
                k0 = i * 16
                idx_b[b][pl.ds(k0, 16)] = votes_b[b][pl.ds(k0, 16)] >> 1
            pltpu.make_async_copy(ssc.at[idx_b[b]], rows_b[b], sem_b[b]).start()
            pltpu.sync_copy(legs_hbm.at[pl.ds(base, CHUNK)], legs_b[b])

        def compute(g, b):
            pltpu.make_async_copy(ssc.at[idx_b[b]], rows_b[b], sem_b[b]).wait()

            @plsc.parallel_loop(0, CHUNK // 16, unroll=8)
            def group_body(i):
                k0 = i * 16
                v = votes_b[b][pl.ds(k0, 16)]
                p2 = (v & 1) * 2
                lv = legs_b[b][pl.ds(k0, 16)]
                g0 = plsc.load_gather(ip_v, [lv * 2])
                g1 = plsc.load_gather(ip_v, [lv * 2 + 1])
                ridx = lanes + k0
                yp0 = plsc.load_gather(rows_b[b], [ridx, p2])
                yp1 = plsc.load_gather(rows_b[b], [ridx, p2 + 1])
                np0 = plsc.load_gather(rows_b[b], [ridx, p2 + 4])
                np1 = plsc.load_gather(rows_b[b], [ridx, p2 + 5])
                dy0 = g0 - yp0
                dy1 = g1 - yp1
                dn0 = g0 - np0
                dn1 = g1 - np1
                dy = w20 * (dy0 * dy0) + w21 * (dy1 * dy1)
                dn = w20 * (dn0 * dn0) + w21 * (dn1 * dn1)
                out_b[b][pl.ds(k0, 16)] = (jnp.exp(-0.5 * dy)
                                           - jnp.exp(-0.5 * dn))
            pltpu.sync_copy(out_b[b], out_hbm.at[pl.ds(wbase + g * CHUNK,
                                                       CHUNK)])

        fetch(0, 0)

        def loop_body(gg, _):
            for b in range(2):
                g = gg * 2 + b

                @pl.when(g + 1 < n_chunks)
                def _():
                    fetch(g + 1, 1 - b)

                compute(g, b)
            return 0

        lax.fori_loop(0, n_chunks // 2, loop_body, 0)

    return impl


def kernel(legs, votes, ideal_points, yes_points, no_points, w):
    M = GBLK * 128
    ymain = yes_points[:M].reshape(GBLK, 128, 2).transpose(0, 2, 1).reshape(-1)
    nmain = no_points[:M].reshape(GBLK, 128, 2).transpose(0, 2, 1).reshape(-1)
    ytail = yes_points[M:].reshape(-1)
    ntail = no_points[M:].reshape(-1)
    wpad = jnp.pad(w.astype(jnp.float32), (0, 14))
    ip_flat = jnp.reshape(ideal_points.astype(jnp.float32), (-1,))
    info = plsc.get_sparse_core_info()
    nw = info.num_cores * info.num_subcores
    impl = _make_impl(B // nw)
    out, _ = impl(legs.astype(jnp.int32), votes.astype(jnp.int32),
                  ip_flat, ymain, ytail, nmain, ntail, wpad)
    return out
